# baseline (speedup 1.0000x reference)
import functools

import jax
import jax.numpy as jnp
from jax import lax
from jax.experimental import pallas as pl

_WAY = 5
_SHOT = 5
_QUERY = 15
_K = 3
_HW = 49
_SEG = _SHOT * _HW
_SEGP = 256
_QB = 15
_RB = _QB * _HW

def _split(x):
    hi = x.astype(jnp.bfloat16)
    lo = (x - hi.astype(jnp.float32)).astype(jnp.bfloat16)
    return hi, lo


def _dot_x2(a, b_exact):
    ah, al = _split(a)
    bb = b_exact.astype(jnp.bfloat16)
    f32 = jnp.float32
    return (jnp.dot(ah, bb, preferred_element_type=f32)
            + jnp.dot(al, bb, preferred_element_type=f32))


def _dot_x3(a, b):
    ah, al = _split(a)
    bh, bl = _split(b)
    f32 = jnp.float32
    return (jnp.dot(ah, bh, preferred_element_type=f32)
            + jnp.dot(ah, bl, preferred_element_type=f32)
            + jnp.dot(al, bh, preferred_element_type=f32))


def _insert(t1, t2, t3, v):
    return (jnp.maximum(t1, v),
            jnp.maximum(t2, jnp.minimum(t1, v)),
            jnp.maximum(t3, jnp.minimum(t2, v)))


def _merge(a, b):
    t = a
    for i in range(3):
        t = _insert(*t, b[i])
    return t


def _dn4_block(q_ref, s_ref, o_ref):
    qt = q_ref[0]
    st = s_ref[...]
    c = qt.shape[0]

    sn2 = jnp.sum(st * st, axis=1, keepdims=True)
    sn = st * (1.0 / jnp.maximum(jnp.sqrt(sn2), 1e-12))

    r_io = lax.broadcasted_iota(jnp.int32, (_RB, _QB), 0)
    j_io = lax.broadcasted_iota(jnp.int32, (_RB, _QB), 1)
    grp = ((r_io >= j_io * _HW) & (r_io < (j_io + 1) * _HW))
    p = grp.astype(jnp.float32)
    g = _dot_x2(qt * qt, p)
    inv = 1.0 / jnp.maximum(jnp.sqrt(g), 1e-12)
    invx = _dot_x2(inv, p.T)
    qn = qt * invx

    rel = _dot_x3(sn, qn)
    rel5 = rel.reshape(_WAY, _SEGP, _RB)

    neg = jnp.float32(-jnp.inf)
    nslab = _SEG // 8 + 1
    v0 = rel5[:, 0:8]
    v1 = rel5[:, 8:16]
    t1 = jnp.maximum(v0, v1)
    t2 = jnp.minimum(v0, v1)
    t3 = jnp.full(t1.shape, neg)
    sub = lax.broadcasted_iota(jnp.int32, (_WAY, 8, _RB), 1)
    for i in range(2, nslab):
        v = rel5[:, 8 * i:8 * i + 8]
        if 8 * i + 8 > _SEG:
            v = jnp.where(sub < _SEG - 8 * i, v, neg)
        t1, t2, t3 = _insert(t1, t2, t3, v)

    trip = (t1, t2, t3)
    width = 8
    while width > 1:
        half = width // 2
        a = tuple(x[:, :half] for x in trip)
        b = tuple(x[:, half:width] for x in trip)
        trip = _merge(a, b)
        width = half
    sum3 = (trip[0] + trip[1] + trip[2])[:, 0, :]

    o_ref[0] = _dot_x2(sum3, p)


@functools.partial(jax.jit, static_argnames=())
def kernel(query_feat, support_feat):
    t, wq, c, h, w = query_feat.shape
    hw = h * w
    nqb = wq // _QB
    outs = []
    for ti in range(t):
        q2 = query_feat[ti].reshape(nqb, _QB, c, hw).transpose(0, 2, 1, 3)
        q2 = q2.reshape(nqb, c, _RB)
        s2 = support_feat[ti].reshape(_WAY, _SHOT, c, hw).transpose(0, 1, 3, 2)
        s2 = s2.reshape(_WAY, _SEG, c)
        s2 = jnp.pad(s2, ((0, 0), (0, _SEGP - _SEG), (0, 0)))
        s2 = s2.reshape(_WAY * _SEGP, c)

        out = pl.pallas_call(
            _dn4_block,
            grid=(1, nqb),
            in_specs=[
                pl.BlockSpec((1, c, _RB), lambda ti_, qi: (qi, 0, 0)),
                pl.BlockSpec((_WAY * _SEGP, c), lambda ti_, qi: (0, 0)),
            ],
            out_specs=pl.BlockSpec((1, _WAY, _QB), lambda ti_, qi: (qi, 0, 0)),
            out_shape=jax.ShapeDtypeStruct((nqb, _WAY, _QB), jnp.float32),
        )(q2, s2)
        outs.append(out.transpose(0, 2, 1).reshape(wq, _WAY))
    return jnp.stack(outs, axis=0)


# speedup vs baseline: 2.7010x; 2.7010x over previous
import functools

import jax
import jax.numpy as jnp
from jax import lax
from jax.experimental import pallas as pl

_WAY = 5
_SHOT = 5
_QUERY = 15
_K = 3
_HW = 49
_SEG = _SHOT * _HW
_SEGP = 256
_QB = 15
_RB = _QB * _HW

def _split(x):
    hi = x.astype(jnp.bfloat16)
    lo = (x - hi.astype(jnp.float32)).astype(jnp.bfloat16)
    return hi, lo


def _dot_x2(a, b_exact):
    ah, al = _split(a)
    bb = b_exact.astype(jnp.bfloat16)
    f32 = jnp.float32
    return (jnp.dot(ah, bb, preferred_element_type=f32)
            + jnp.dot(al, bb, preferred_element_type=f32))


def _dot_x3(a, b):
    ah, al = _split(a)
    bh, bl = _split(b)
    f32 = jnp.float32
    return (jnp.dot(ah, bh, preferred_element_type=f32)
            + jnp.dot(ah, bl, preferred_element_type=f32)
            + jnp.dot(al, bh, preferred_element_type=f32))


def _insert(t1, t2, t3, v):
    return (jnp.maximum(t1, v),
            jnp.maximum(t2, jnp.minimum(t1, v)),
            jnp.maximum(t3, jnp.minimum(t2, v)))


def _merge(a, b):
    t = a
    for i in range(3):
        t = _insert(*t, b[i])
    return t


def _dn4_block(q_ref, s_ref, o_ref):
    qt = q_ref[0, 0]
    st = s_ref[0]
    c = qt.shape[0]

    sn2 = jnp.sum(st * st, axis=1, keepdims=True)
    sn = st * (1.0 / jnp.maximum(jnp.sqrt(sn2), 1e-12))

    r_io = lax.broadcasted_iota(jnp.int32, (_RB, _QB), 0)
    j_io = lax.broadcasted_iota(jnp.int32, (_RB, _QB), 1)
    grp = ((r_io >= j_io * _HW) & (r_io < (j_io + 1) * _HW))
    p = grp.astype(jnp.float32)
    g = _dot_x2(qt * qt, p)
    inv = 1.0 / jnp.maximum(jnp.sqrt(g), 1e-12)
    invx = _dot_x2(inv, p.T)
    qn = qt * invx

    rel = jnp.dot(sn.astype(jnp.bfloat16), qn.astype(jnp.bfloat16),
                  preferred_element_type=jnp.float32)
    rel5 = rel.reshape(_WAY, _SEGP, _RB)

    neg = jnp.float32(-jnp.inf)
    nslab = _SEG // 8 + 1
    v0 = rel5[:, 0:8]
    v1 = rel5[:, 8:16]
    t1 = jnp.maximum(v0, v1)
    t2 = jnp.minimum(v0, v1)
    t3 = jnp.full(t1.shape, neg)
    sub = lax.broadcasted_iota(jnp.int32, (_WAY, 8, _RB), 1)
    for i in range(2, nslab):
        v = rel5[:, 8 * i:8 * i + 8]
        if 8 * i + 8 > _SEG:
            v = jnp.where(sub < _SEG - 8 * i, v, neg)
        t1, t2, t3 = _insert(t1, t2, t3, v)

    trip = (t1, t2, t3)
    width = 8
    while width > 1:
        half = width // 2
        a = tuple(x[:, :half] for x in trip)
        b = tuple(x[:, half:width] for x in trip)
        trip = _merge(a, b)
        width = half
    sum3 = (trip[0] + trip[1] + trip[2])[:, 0, :]

    o_ref[0, 0] = _dot_x2(sum3, p)


@functools.partial(jax.jit, static_argnames=())
def kernel(query_feat, support_feat):
    t, wq, c, h, w = query_feat.shape
    hw = h * w
    nqb = wq // _QB
    q2 = query_feat.reshape(t, nqb, _QB, c, hw).transpose(0, 1, 3, 2, 4)
    q2 = q2.reshape(t, nqb, c, _RB)
    s2 = support_feat.reshape(t, _WAY, _SHOT, c, hw).transpose(0, 1, 2, 4, 3)
    s2 = s2.reshape(t, _WAY, _SEG, c)
    s2 = jnp.pad(s2, ((0, 0), (0, 0), (0, _SEGP - _SEG), (0, 0)))
    s2 = s2.reshape(t, _WAY * _SEGP, c)

    out = pl.pallas_call(
        _dn4_block,
        grid=(t, nqb),
        in_specs=[
            pl.BlockSpec((1, 1, c, _RB), lambda ti, qi: (ti, qi, 0, 0)),
            pl.BlockSpec((1, _WAY * _SEGP, c), lambda ti, qi: (ti, 0, 0)),
        ],
        out_specs=pl.BlockSpec((1, 1, _WAY, _QB), lambda ti, qi: (ti, qi, 0, 0)),
        out_shape=jax.ShapeDtypeStruct((t, nqb, _WAY, _QB), jnp.float32),
    )(q2, s2)
    return out.transpose(0, 1, 3, 2).reshape(t, wq, _WAY)


# speedup vs baseline: 2.9121x; 1.0782x over previous
import functools

import jax
import jax.numpy as jnp
from jax import lax
from jax.experimental import pallas as pl

_WAY = 5
_SHOT = 5
_QUERY = 15
_K = 3
_HW = 49
_SEG = _SHOT * _HW
_SEGP = 256
_QB = 15
_RB = _QB * _HW


def _split(x):
    hi = x.astype(jnp.bfloat16)
    lo = (x - hi.astype(jnp.float32)).astype(jnp.bfloat16)
    return hi, lo


def _dot_x2(a, b_exact):
    ah, al = _split(a)
    bb = b_exact.astype(jnp.bfloat16)
    f32 = jnp.float32
    return (jnp.dot(ah, bb, preferred_element_type=f32)
            + jnp.dot(al, bb, preferred_element_type=f32))


def _insert(t1, t2, t3, v):
    return (jnp.maximum(t1, v),
            jnp.maximum(t2, jnp.minimum(t1, v)),
            jnp.maximum(t3, jnp.minimum(t2, v)))


def _merge(a, b):
    t = a
    for i in range(3):
        t = _insert(*t, b[i])
    return t


def _dn4_block(q_ref, s_ref, o_ref):
    qt16 = q_ref[0, 0]
    st16 = s_ref[0]
    qt = qt16.astype(jnp.float32)
    st = st16.astype(jnp.float32)

    sn2 = jnp.sum(st * st, axis=1, keepdims=True)
    inv_s = 1.0 / jnp.maximum(jnp.sqrt(sn2), 1e-12)

    r_io = lax.broadcasted_iota(jnp.int32, (_RB, _QB), 0)
    j_io = lax.broadcasted_iota(jnp.int32, (_RB, _QB), 1)
    grp = ((r_io >= j_io * _HW) & (r_io < (j_io + 1) * _HW))
    p = grp.astype(jnp.float32)
    g = _dot_x2(qt * qt, p)
    inv = 1.0 / jnp.maximum(jnp.sqrt(g), 1e-12)
    invx = _dot_x2(inv, p.T)
    qn = (qt * invx).astype(jnp.bfloat16)

    rel = jnp.dot(st16, qn, preferred_element_type=jnp.float32)
    rel = rel * inv_s
    rel5 = rel.reshape(_WAY, _SEGP, _RB)

    neg = jnp.float32(-jnp.inf)
    nslab = _SEG // 8 + 1
    v0 = rel5[:, 0:8]
    v1 = rel5[:, 8:16]
    t1 = jnp.maximum(v0, v1)
    t2 = jnp.minimum(v0, v1)
    t3 = jnp.full(t1.shape, neg)
    sub = lax.broadcasted_iota(jnp.int32, (_WAY, 8, _RB), 1)
    for i in range(2, nslab):
        v = rel5[:, 8 * i:8 * i + 8]
        if 8 * i + 8 > _SEG:
            v = jnp.where(sub < _SEG - 8 * i, v, neg)
        t1, t2, t3 = _insert(t1, t2, t3, v)

    trip = (t1, t2, t3)
    width = 8
    while width > 1:
        half = width // 2
        a = tuple(x[:, :half] for x in trip)
        b = tuple(x[:, half:width] for x in trip)
        trip = _merge(a, b)
        width = half
    sum3 = (trip[0] + trip[1] + trip[2])[:, 0, :]

    o_ref[0, 0] = _dot_x2(sum3, p)


@functools.partial(jax.jit, static_argnames=())
def kernel(query_feat, support_feat):
    t, wq, c, h, w = query_feat.shape
    hw = h * w
    nqb = wq // _QB
    qb16 = query_feat.astype(jnp.bfloat16)
    sb16 = support_feat.astype(jnp.bfloat16)
    q2 = qb16.reshape(t, nqb, _QB, c, hw).transpose(0, 1, 3, 2, 4)
    q2 = q2.reshape(t, nqb, c, _RB)
    s2 = sb16.reshape(t, _WAY, _SHOT, c, hw).transpose(0, 1, 2, 4, 3)
    s2 = s2.reshape(t, _WAY, _SEG, c)
    s2 = jnp.pad(s2, ((0, 0), (0, 0), (0, _SEGP - _SEG), (0, 0)))
    s2 = s2.reshape(t, _WAY * _SEGP, c)

    out = pl.pallas_call(
        _dn4_block,
        grid=(t, nqb),
        in_specs=[
            pl.BlockSpec((1, 1, c, _RB), lambda ti, qi: (ti, qi, 0, 0)),
            pl.BlockSpec((1, _WAY * _SEGP, c), lambda ti, qi: (ti, 0, 0)),
        ],
        out_specs=pl.BlockSpec((1, 1, _WAY, _QB), lambda ti, qi: (ti, qi, 0, 0)),
        out_shape=jax.ShapeDtypeStruct((t, nqb, _WAY, _QB), jnp.float32),
    )(q2, s2)
    return out.transpose(0, 1, 3, 2).reshape(t, wq, _WAY)
